# baseline (speedup 1.0000x reference)
import functools
import math

import jax
import jax.numpy as jnp
from jax import lax
from jax.experimental import pallas as pl
from jax.experimental.pallas import tpu as pltpu
from jax.experimental.pallas import tpu_sc as plsc

SPEC_LEN = 3200
PATCH = 8
D_MODEL = 512
VOCAB = 100
BATCH = 1024
NPOOL = SPEC_LEN // PATCH
SCALE = math.sqrt(float(D_MODEL))

NC, NS = 2, 16
NW = NC * NS
ROWS_PER_W = BATCH // NW
CH = 80
NCH = NPOOL // CH
GROUPS = NPOOL // 16
G_PER_CH = CH // 16


def _scale_body(t_ref, o_ref):
    o_ref[...] = t_ref[...] * SCALE


def _sc_body(spec_hbm, table_hbm, out_hbm, spec_v, idx_v, buf_v, sem):
    wid = lax.axis_index("s") * NC + lax.axis_index("c")
    iota = lax.iota(jnp.int32, 16)

    def row_body(r, carry):
        b = wid * ROWS_PER_W + r
        pltpu.sync_copy(spec_hbm.at[b], spec_v)
        for g in range(GROUPS):
            base = g * 128 + iota * PATCH
            m = plsc.load_gather(spec_v, [base])
            for j in range(1, PATCH):
                m = jnp.maximum(m, plsc.load_gather(spec_v, [base + j]))
            idx_v[g // G_PER_CH, pl.ds((g % G_PER_CH) * 16, 16)] = m.astype(
                jnp.int32
            )

        def ch_body(k, carry2):
            pltpu.async_copy(table_hbm.at[idx_v.at[k]], buf_v, sem).wait()
            pltpu.sync_copy(buf_v, out_hbm.at[pl.ds(b * NPOOL + k * CH, CH)])
            return carry2

        return lax.fori_loop(0, NCH, ch_body, carry)

    lax.fori_loop(0, ROWS_PER_W, row_body, 0)


def kernel(spec, embed_table):
    scaled = pl.pallas_call(
        _scale_body,
        out_shape=jax.ShapeDtypeStruct((VOCAB, D_MODEL), jnp.float32),
    )(embed_table)
    spec2 = spec.reshape(BATCH, SPEC_LEN)

    mesh = plsc.VectorSubcoreMesh(core_axis_name="c", subcore_axis_name="s")
    sc = pl.kernel(
        _sc_body,
        out_type=jax.ShapeDtypeStruct((BATCH * NPOOL, D_MODEL), jnp.float32),
        mesh=mesh,
        scratch_types=[
            pltpu.VMEM((SPEC_LEN,), jnp.float32),
            pltpu.VMEM((NCH, CH), jnp.int32),
            pltpu.VMEM((CH, D_MODEL), jnp.float32),
            pltpu.SemaphoreType.DMA,
        ],
        compiler_params=pltpu.CompilerParams(needs_layout_passes=False),
    )
    out = sc(spec2, scaled)
    return out.reshape(BATCH, NPOOL, D_MODEL)
